# baseline (speedup 1.0000x reference)
import functools

import jax
import jax.numpy as jnp
from jax import lax
from jax.experimental import pallas as pl
from jax.experimental.pallas import tpu as pltpu
from jax.experimental.pallas import tpu_sc as plsc

_NC, _NS = 2, 16
_NW = _NC * _NS
_L = 16
_NBUF = 4


def _rank_body(x_ref, out_ref, *, n, bb):
    x = x_ref[...]
    xi = x[:, :, None]
    xj = x[:, None, :]
    lt = xj < xi
    le = xj <= xi
    ii = lax.broadcasted_iota(jnp.int32, (bb, n, n), 1)
    jj = lax.broadcasted_iota(jnp.int32, (bb, n, n), 2)
    c = jnp.where(lt | ((jj < ii) & le), 1.0, 0.0)
    out_ref[...] = jnp.sum(c, axis=-1).astype(jnp.int32)


def _rank_call(numbers, bb=8, interpret=False):
    b, n = numbers.shape
    return pl.pallas_call(
        functools.partial(_rank_body, n=n, bb=bb),
        grid=(b // bb,),
        in_specs=[pl.BlockSpec((bb, n), lambda i: (i, 0))],
        out_specs=pl.BlockSpec((bb, n), lambda i: (i, 0)),
        out_shape=jax.ShapeDtypeStruct((b, n), jnp.int32),
        interpret=interpret,
    )(numbers)


def _make_sc_lookup(b, n, d):
    rpw = b // _NW
    nk = (n + _L - 1) // _L
    rem = n - (nk - 1) * _L
    mesh = plsc.VectorSubcoreMesh(
        core_axis_name="c", subcore_axis_name="s",
        num_cores=_NC, num_subcores=_NS,
    )

    @functools.partial(
        pl.kernel,
        out_type=jax.ShapeDtypeStruct((n, d, b), jnp.float32),
        mesh=mesh,
        compiler_params=pltpu.CompilerParams(needs_layout_passes=False),
        scratch_types=[
            pltpu.VMEM((n * d,), jnp.float32),
            pltpu.VMEM((rpw * n + _L,), jnp.int32),
            pltpu.VMEM((n * rpw,), jnp.int32),
            pltpu.VMEM((_NBUF, d, rpw), jnp.float32),
            pltpu.SemaphoreType.DMA,
        ],
    )
    def sc_lookup(rank1, tbl1, out_t, tbl_v, rank_v, sidx_v, blk_v, sem):
        wid = lax.axis_index("s") * _NC + lax.axis_index("c")
        bbase = wid * rpw
        pltpu.sync_copy(tbl1.at[pl.ds(0, n * d)], tbl_v.at[pl.ds(0, n * d)])
        pltpu.sync_copy(
            rank1.at[pl.ds(bbase * n, rpw * n)], rank_v.at[pl.ds(0, rpw * n)]
        )

        iota16 = lax.iota(jnp.int32, _L)

        def inv_body(beta, carry):
            for k in range(nk):
                vals = rank_v[pl.ds(beta * n + k * _L, _L)]
                idx = vals * rpw + beta
                ivec = iota16 + (k * _L)
                if k < nk - 1:
                    plsc.store_scatter(sidx_v, [idx], ivec)
                else:
                    plsc.store_scatter(sidx_v, [idx], ivec, mask=iota16 < rem)
            return carry

        lax.fori_loop(0, rpw, inv_body, 0)

        def drain_one():
            pltpu.make_async_copy(
                out_t.at[0, :, pl.ds(0, rpw)], blk_v.at[0], sem
            ).wait()

        def j_body(j, carry):
            buf = lax.rem(j, _NBUF)
            pl.when(j >= _NBUF)(drain_one)
            base = [
                sidx_v[pl.ds(j * rpw + m * _L, _L)] * d for m in range(rpw // _L)
            ]

            for c in range(d):
                for m in range(rpw // _L):
                    vals = plsc.load_gather(tbl_v, [base[m] + c])
                    blk_v[buf, c, pl.ds(m * _L, _L)] = vals
            pltpu.async_copy(
                blk_v.at[buf], out_t.at[j, :, pl.ds(bbase, rpw)], sem
            )
            return carry

        lax.fori_loop(0, n, j_body, 0)
        for _ in range(_NBUF):
            drain_one()

    return sc_lookup


def kernel(numbers, table):
    b, n = numbers.shape
    _, d = table.shape
    rank = _rank_call(numbers)
    rank1 = rank.reshape(b * n)
    tbl1 = table[:n].reshape(n * d)
    out_t = _make_sc_lookup(b, n, d)(rank1, tbl1)
    return jnp.transpose(out_t, (2, 0, 1))


# speedup vs baseline: 1.6415x; 1.6415x over previous
import functools

import jax
import jax.numpy as jnp
from jax import lax
from jax.experimental import pallas as pl
from jax.experimental.pallas import tpu as pltpu
from jax.experimental.pallas import tpu_sc as plsc

_NC, _NS = 2, 16
_NW = _NC * _NS
_L = 16
_NBUF = 4


def _rank_body(x_ref, out_ref, *, n, bb):
    x = x_ref[...]
    xi = x[:, :, None]
    xj = x[:, None, :]
    lt = xj < xi
    le = xj <= xi
    ii = lax.broadcasted_iota(jnp.int32, (bb, n, n), 1)
    jj = lax.broadcasted_iota(jnp.int32, (bb, n, n), 2)
    c = jnp.where(lt | ((jj < ii) & le), 1.0, 0.0)
    out_ref[...] = jnp.sum(c, axis=-1).astype(jnp.int32)


def _rank_call(numbers, bb=8, interpret=False):
    b, n = numbers.shape
    return pl.pallas_call(
        functools.partial(_rank_body, n=n, bb=bb),
        grid=(b // bb,),
        in_specs=[pl.BlockSpec((bb, n), lambda i: (i, 0))],
        out_specs=pl.BlockSpec((bb, n), lambda i: (i, 0)),
        out_shape=jax.ShapeDtypeStruct((b, n), jnp.int32),
        interpret=interpret,
    )(numbers)


def _make_sc_lookup(b, n, d):
    rpw = b // _NW
    nk = (n + _L - 1) // _L
    rem = n - (nk - 1) * _L
    mesh = plsc.VectorSubcoreMesh(
        core_axis_name="c", subcore_axis_name="s",
        num_cores=_NC, num_subcores=_NS,
    )

    ts = d + 1
    ss = rpw + 1

    @functools.partial(
        pl.kernel,
        out_type=jax.ShapeDtypeStruct((n, d, b), jnp.float32),
        mesh=mesh,
        compiler_params=pltpu.CompilerParams(needs_layout_passes=False),
        scratch_types=[
            pltpu.VMEM((n * d,), jnp.float32),
            pltpu.VMEM((n * ts,), jnp.float32),
            pltpu.VMEM((rpw * n + _L,), jnp.int32),
            pltpu.VMEM((n * ss,), jnp.int32),
            pltpu.VMEM((_NBUF, d, rpw), jnp.float32),
            pltpu.SemaphoreType.DMA,
        ],
    )
    def sc_lookup(rank1, tbl1, out_t, tbl0_v, tbl_v, rank_v, sidx_v, blk_v, sem):
        wid = lax.axis_index("s") * _NC + lax.axis_index("c")
        bbase = wid * rpw
        pltpu.sync_copy(tbl1.at[pl.ds(0, n * d)], tbl0_v.at[pl.ds(0, n * d)])
        pltpu.sync_copy(
            rank1.at[pl.ds(bbase * n, rpw * n)], rank_v.at[pl.ds(0, rpw * n)]
        )

        def relay_body(r, carry):
            for h in range(d // _L):
                tbl_v[pl.ds(r * ts + h * _L, _L)] = tbl0_v[
                    pl.ds(r * d + h * _L, _L)
                ]
            return carry

        lax.fori_loop(0, n, relay_body, 0)

        iota16 = lax.iota(jnp.int32, _L)

        def inv_body(beta, carry):
            for k in range(nk):
                vals = rank_v[pl.ds(beta * n + k * _L, _L)]
                idx = vals * ss + beta
                ivec = iota16 + (k * _L)
                if k < nk - 1:
                    plsc.store_scatter(sidx_v, [idx], ivec)
                else:
                    plsc.store_scatter(sidx_v, [idx], ivec, mask=iota16 < rem)
            return carry

        lax.fori_loop(0, rpw, inv_body, 0)

        def drain_one():
            pltpu.make_async_copy(
                out_t.at[0, :, pl.ds(0, rpw)], blk_v.at[0], sem
            ).wait()

        def j_body(j, carry):
            buf = lax.rem(j, _NBUF)
            pl.when(j >= _NBUF)(drain_one)
            base = [
                sidx_v[pl.ds(j * ss + m * _L, _L)] * ts for m in range(rpw // _L)
            ]

            for c in range(d):
                for m in range(rpw // _L):
                    vals = plsc.load_gather(tbl_v, [base[m] + c])
                    blk_v[buf, c, pl.ds(m * _L, _L)] = vals
            pltpu.async_copy(
                blk_v.at[buf], out_t.at[j, :, pl.ds(bbase, rpw)], sem
            )
            return carry

        lax.fori_loop(0, n, j_body, 0)
        for _ in range(_NBUF):
            drain_one()

    return sc_lookup


def kernel(numbers, table):
    b, n = numbers.shape
    _, d = table.shape
    rank = _rank_call(numbers)
    rank1 = rank.reshape(b * n)
    tbl1 = table[:n].reshape(n * d)
    out_t = _make_sc_lookup(b, n, d)(rank1, tbl1)
    return jnp.transpose(out_t, (2, 0, 1))


# speedup vs baseline: 2.7917x; 1.7007x over previous
import functools

import jax
import jax.numpy as jnp
from jax import lax
from jax.experimental import pallas as pl
from jax.experimental.pallas import tpu as pltpu
from jax.experimental.pallas import tpu_sc as plsc

_NC, _NS = 2, 16
_NW = _NC * _NS
_L = 16
_NBUF = 4


def _rank_body(x_ref, out_ref, *, n, grp):
    zeros = jnp.zeros((8, 128), jnp.float32)
    for g in range(n // grp):
        i0 = g * grp
        xi = [x_ref[i0 + t] for t in range(grp)]

        def le_body(j, accs, xi=xi):
            xj = x_ref[j]
            return tuple(
                a + jnp.where(xj <= v, 1.0, 0.0) for a, v in zip(accs, xi)
            )

        def lt_body(j, accs, xi=xi):
            xj = x_ref[j]
            return tuple(
                a + jnp.where(xj < v, 1.0, 0.0) for a, v in zip(accs, xi)
            )

        accs = lax.fori_loop(0, i0, le_body, (zeros,) * grp, unroll=2)
        accs = list(lax.fori_loop(i0 + grp, n, lt_body, accs, unroll=2))
        for jj in range(grp):
            for t in range(grp):
                if jj == t:
                    continue
                cm = (xi[jj] <= xi[t]) if jj < t else (xi[jj] < xi[t])
                accs[t] = accs[t] + jnp.where(cm, 1.0, 0.0)
        for t in range(grp):
            out_ref[i0 + t] = accs[t].astype(jnp.int32)


def _rank_call(numbers, grp=8, interpret=False):
    n, bh, _ = numbers.shape
    return pl.pallas_call(
        functools.partial(_rank_body, n=n, grp=grp),
        grid=(bh // 8,),
        in_specs=[pl.BlockSpec((n, 8, 128), lambda i: (0, i, 0))],
        out_specs=pl.BlockSpec((n, 8, 128), lambda i: (0, i, 0)),
        out_shape=jax.ShapeDtypeStruct((n, bh, 128), jnp.int32),
        interpret=interpret,
    )(numbers)


def _make_sc_lookup(b, n, d):
    rpw = b // _NW
    nk = (n + _L - 1) // _L
    rem = n - (nk - 1) * _L
    mesh = plsc.VectorSubcoreMesh(
        core_axis_name="c", subcore_axis_name="s",
        num_cores=_NC, num_subcores=_NS,
    )

    ts = d + 1
    ss = rpw + 1

    @functools.partial(
        pl.kernel,
        out_type=jax.ShapeDtypeStruct((n, d, b), jnp.float32),
        mesh=mesh,
        compiler_params=pltpu.CompilerParams(needs_layout_passes=False),
        scratch_types=[
            pltpu.VMEM((n * d,), jnp.float32),
            pltpu.VMEM((n * ts,), jnp.float32),
            pltpu.VMEM((rpw * n + _L,), jnp.int32),
            pltpu.VMEM((n * ss,), jnp.int32),
            pltpu.VMEM((_NBUF, d, rpw), jnp.float32),
            pltpu.SemaphoreType.DMA,
        ],
    )
    def sc_lookup(rank1, tbl1, out_t, tbl0_v, tbl_v, rank_v, sidx_v, blk_v, sem):
        wid = lax.axis_index("s") * _NC + lax.axis_index("c")
        bbase = wid * rpw
        pltpu.sync_copy(tbl1.at[pl.ds(0, n * d)], tbl0_v.at[pl.ds(0, n * d)])
        pltpu.sync_copy(
            rank1.at[pl.ds(bbase * n, rpw * n)], rank_v.at[pl.ds(0, rpw * n)]
        )

        def relay_body(r, carry):
            for h in range(d // _L):
                tbl_v[pl.ds(r * ts + h * _L, _L)] = tbl0_v[
                    pl.ds(r * d + h * _L, _L)
                ]
            return carry

        lax.fori_loop(0, n, relay_body, 0)

        iota16 = lax.iota(jnp.int32, _L)

        def inv_body(beta, carry):
            for k in range(nk):
                vals = rank_v[pl.ds(beta * n + k * _L, _L)]
                idx = vals * ss + beta
                ivec = iota16 + (k * _L)
                if k < nk - 1:
                    plsc.store_scatter(sidx_v, [idx], ivec)
                else:
                    plsc.store_scatter(sidx_v, [idx], ivec, mask=iota16 < rem)
            return carry

        lax.fori_loop(0, rpw, inv_body, 0)

        def drain_one():
            pltpu.make_async_copy(
                out_t.at[0, :, pl.ds(0, rpw)], blk_v.at[0], sem
            ).wait()

        def j_body(j, carry):
            buf = lax.rem(j, _NBUF)
            pl.when(j >= _NBUF)(drain_one)
            base = [
                sidx_v[pl.ds(j * ss + m * _L, _L)] * ts for m in range(rpw // _L)
            ]

            for c in range(d):
                for m in range(rpw // _L):
                    vals = plsc.load_gather(tbl_v, [base[m] + c])
                    blk_v[buf, c, pl.ds(m * _L, _L)] = vals
            pltpu.async_copy(
                blk_v.at[buf], out_t.at[j, :, pl.ds(bbase, rpw)], sem
            )
            return carry

        lax.fori_loop(0, n, j_body, 0)
        for _ in range(_NBUF):
            drain_one()

    return sc_lookup


def kernel(numbers, table):
    b, n = numbers.shape
    _, d = table.shape
    x_t = jnp.transpose(numbers).reshape(n, b // 128, 128)
    rank_t = _rank_call(x_t)
    rank1 = jnp.transpose(rank_t.reshape(n, b)).reshape(b * n)
    tbl1 = table[:n].reshape(n * d)
    out_t = _make_sc_lookup(b, n, d)(rank1, tbl1)
    return jnp.transpose(out_t, (2, 0, 1))


# speedup vs baseline: 4.6824x; 1.6773x over previous
import functools

import jax
import jax.numpy as jnp
from jax import lax
from jax.experimental import pallas as pl
from jax.experimental.pallas import tpu as pltpu
from jax.experimental.pallas import tpu_sc as plsc

_NC, _NS = 2, 16
_NW = _NC * _NS
_L = 16
_NBUF = 4


def _rank_body(x_ref, out_ref, *, n, grp):
    zeros = jnp.zeros((8, 128), jnp.float32)
    for g in range(n // grp):
        i0 = g * grp
        xi = [x_ref[i0 + t] for t in range(grp)]

        def le_body(j, accs, xi=xi):
            xj = x_ref[j]
            return tuple(
                a + jnp.where(xj <= v, 1.0, 0.0) for a, v in zip(accs, xi)
            )

        def lt_body(j, accs, xi=xi):
            xj = x_ref[j]
            return tuple(
                a + jnp.where(xj < v, 1.0, 0.0) for a, v in zip(accs, xi)
            )

        accs = lax.fori_loop(0, i0, le_body, (zeros,) * grp, unroll=2)
        accs = list(lax.fori_loop(i0 + grp, n, lt_body, accs, unroll=2))
        for jj in range(grp):
            for t in range(grp):
                if jj == t:
                    continue
                cm = (xi[jj] <= xi[t]) if jj < t else (xi[jj] < xi[t])
                accs[t] = accs[t] + jnp.where(cm, 1.0, 0.0)
        for t in range(grp):
            out_ref[i0 + t] = accs[t].astype(jnp.int32)


def _rank_call(numbers, grp=8, interpret=False):
    n, bh, _ = numbers.shape
    return pl.pallas_call(
        functools.partial(_rank_body, n=n, grp=grp),
        grid=(bh // 8,),
        in_specs=[pl.BlockSpec((n, 8, 128), lambda i: (0, i, 0))],
        out_specs=pl.BlockSpec((n, 8, 128), lambda i: (0, i, 0)),
        out_shape=jax.ShapeDtypeStruct((n, bh, 128), jnp.int32),
        interpret=interpret,
    )(numbers)


def _make_sc_lookup(b, n, d):
    rpw = b // _NW
    nk = (n + _L - 1) // _L
    rem = n - (nk - 1) * _L
    mesh = plsc.VectorSubcoreMesh(
        core_axis_name="c", subcore_axis_name="s",
        num_cores=_NC, num_subcores=_NS,
    )

    ts = d + 1
    ss = rpw + 1

    @functools.partial(
        pl.kernel,
        out_type=jax.ShapeDtypeStruct((n, d, b), jnp.float32),
        mesh=mesh,
        compiler_params=pltpu.CompilerParams(needs_layout_passes=False),
        scratch_types=[
            pltpu.VMEM((n * d,), jnp.float32),
            pltpu.VMEM((n * ts,), jnp.float32),
            pltpu.VMEM((rpw * n + _L,), jnp.int32),
            pltpu.VMEM((n * ss,), jnp.int32),
            pltpu.VMEM((_NBUF, d, rpw), jnp.float32),
            pltpu.SemaphoreType.DMA,
        ],
    )
    def sc_lookup(rank1, tbl1, out_t, tbl0_v, tbl_v, rank_v, sidx_v, blk_v, sem):
        wid = lax.axis_index("s") * _NC + lax.axis_index("c")
        bbase = wid * rpw
        pltpu.sync_copy(tbl1.at[pl.ds(0, n * d)], tbl0_v.at[pl.ds(0, n * d)])
        pltpu.sync_copy(
            rank1.at[pl.ds(bbase * n, rpw * n)], rank_v.at[pl.ds(0, rpw * n)]
        )

        def relay_body(r, carry):
            for h in range(d // _L):
                tbl_v[pl.ds(r * ts + h * _L, _L)] = tbl0_v[
                    pl.ds(r * d + h * _L, _L)
                ]
            return carry

        lax.fori_loop(0, n, relay_body, 0)

        iota16 = lax.iota(jnp.int32, _L)

        def inv_body(beta, carry):
            for k in range(nk):
                vals = rank_v[pl.ds(beta * n + k * _L, _L)]
                idx = vals * ss + beta
                ivec = iota16 + (k * _L)
                if k < nk - 1:
                    plsc.store_scatter(sidx_v, [idx], ivec)
                else:
                    plsc.store_scatter(sidx_v, [idx], ivec, mask=iota16 < rem)
            return carry

        lax.fori_loop(0, rpw, inv_body, 0)

        def drain_one():
            pltpu.make_async_copy(
                out_t.at[0, :, pl.ds(0, rpw)], blk_v.at[0], sem
            ).wait()

        def j_body(j, carry):
            buf = lax.rem(j, _NBUF)
            pl.when(j >= _NBUF)(drain_one)
            base = [
                sidx_v[pl.ds(j * ss + m * _L, _L)] * ts for m in range(rpw // _L)
            ]

            for c0 in range(0, d, 4):
                vals = [
                    (c0 + dc, m, plsc.load_gather(tbl_v, [base[m] + c0 + dc]))
                    for dc in range(4)
                    for m in range(rpw // _L)
                ]
                for c, m, v in vals:
                    blk_v[buf, c, pl.ds(m * _L, _L)] = v
            pltpu.async_copy(
                blk_v.at[buf], out_t.at[j, :, pl.ds(bbase, rpw)], sem
            )
            return carry

        lax.fori_loop(0, n, j_body, 0)
        for _ in range(_NBUF):
            drain_one()

    return sc_lookup


def kernel(numbers, table):
    b, n = numbers.shape
    _, d = table.shape
    x_t = jnp.transpose(numbers).reshape(n, b // 128, 128)
    rank_t = _rank_call(x_t)
    rank1 = jnp.transpose(rank_t.reshape(n, b)).reshape(b * n)
    tbl1 = table[:n].reshape(n * d)
    out_t = _make_sc_lookup(b, n, d)(rank1, tbl1)
    return jnp.transpose(out_t, (2, 0, 1))


# speedup vs baseline: 5.1011x; 1.0894x over previous
import functools

import jax
import jax.numpy as jnp
from jax import lax
from jax.experimental import pallas as pl
from jax.experimental.pallas import tpu as pltpu
from jax.experimental.pallas import tpu_sc as plsc

_NC, _NS = 2, 16
_NW = _NC * _NS
_L = 16
_NBUF = 6


def _rank_body(x_ref, out_ref, *, n, grp):
    zeros = jnp.zeros((8, 128), jnp.float32)
    for g in range(n // grp):
        i0 = g * grp
        xi = [x_ref[i0 + t] for t in range(grp)]

        def le_body(j, accs, xi=xi):
            xj = x_ref[j]
            return tuple(
                a + jnp.where(xj <= v, 1.0, 0.0) for a, v in zip(accs, xi)
            )

        def lt_body(j, accs, xi=xi):
            xj = x_ref[j]
            return tuple(
                a + jnp.where(xj < v, 1.0, 0.0) for a, v in zip(accs, xi)
            )

        accs = lax.fori_loop(0, i0, le_body, (zeros,) * grp, unroll=4)
        accs = list(lax.fori_loop(i0 + grp, n, lt_body, accs, unroll=4))
        for jj in range(grp):
            for t in range(grp):
                if jj == t:
                    continue
                cm = (xi[jj] <= xi[t]) if jj < t else (xi[jj] < xi[t])
                accs[t] = accs[t] + jnp.where(cm, 1.0, 0.0)
        for t in range(grp):
            out_ref[i0 + t] = accs[t].astype(jnp.int32)


def _rank_call(numbers, grp=8, interpret=False):
    n, bh, _ = numbers.shape
    return pl.pallas_call(
        functools.partial(_rank_body, n=n, grp=grp),
        grid=(bh // 8,),
        in_specs=[pl.BlockSpec((n, 8, 128), lambda i: (0, i, 0))],
        out_specs=pl.BlockSpec((n, 8, 128), lambda i: (0, i, 0)),
        out_shape=jax.ShapeDtypeStruct((n, bh, 128), jnp.int32),
        interpret=interpret,
    )(numbers)


def _make_sc_lookup(b, n, d):
    rpw = b // _NW
    nk = (n + _L - 1) // _L
    rem = n - (nk - 1) * _L
    mesh = plsc.VectorSubcoreMesh(
        core_axis_name="c", subcore_axis_name="s",
        num_cores=_NC, num_subcores=_NS,
    )

    ts = d + 1
    ss = rpw + 1

    @functools.partial(
        pl.kernel,
        out_type=jax.ShapeDtypeStruct((n, d, b), jnp.float32),
        mesh=mesh,
        compiler_params=pltpu.CompilerParams(needs_layout_passes=False),
        scratch_types=[
            pltpu.VMEM((n * d,), jnp.float32),
            pltpu.VMEM((n * ts,), jnp.float32),
            pltpu.VMEM((rpw * n + _L,), jnp.int32),
            pltpu.VMEM((n * ss,), jnp.int32),
            pltpu.VMEM((_NBUF, d, rpw), jnp.float32),
            pltpu.SemaphoreType.DMA,
        ],
    )
    def sc_lookup(rank1, tbl1, out_t, tbl0_v, tbl_v, rank_v, sidx_v, blk_v, sem):
        wid = lax.axis_index("s") * _NC + lax.axis_index("c")
        bbase = wid * rpw
        pltpu.sync_copy(tbl1.at[pl.ds(0, n * d)], tbl0_v.at[pl.ds(0, n * d)])
        pltpu.sync_copy(
            rank1.at[pl.ds(bbase * n, rpw * n)], rank_v.at[pl.ds(0, rpw * n)]
        )

        def relay_body(r, carry):
            for h in range(d // _L):
                tbl_v[pl.ds(r * ts + h * _L, _L)] = tbl0_v[
                    pl.ds(r * d + h * _L, _L)
                ]
            return carry

        lax.fori_loop(0, n, relay_body, 0)

        iota16 = lax.iota(jnp.int32, _L)

        def inv_body(beta, carry):
            for k in range(nk):
                vals = rank_v[pl.ds(beta * n + k * _L, _L)]
                idx = vals * ss + beta
                ivec = iota16 + (k * _L)
                if k < nk - 1:
                    plsc.store_scatter(sidx_v, [idx], ivec)
                else:
                    plsc.store_scatter(sidx_v, [idx], ivec, mask=iota16 < rem)
            return carry

        lax.fori_loop(0, rpw, inv_body, 0)

        def drain_one():
            pltpu.make_async_copy(
                out_t.at[0, :, pl.ds(0, rpw)], blk_v.at[0], sem
            ).wait()

        def j_body(j, carry):
            buf = lax.rem(j, _NBUF)
            pl.when(j >= _NBUF)(drain_one)
            base = [
                sidx_v[pl.ds(j * ss + m * _L, _L)] * ts for m in range(rpw // _L)
            ]

            for c0 in range(0, d, 4):
                vals = [
                    (c0 + dc, m, plsc.load_gather(tbl_v, [base[m] + c0 + dc]))
                    for dc in range(4)
                    for m in range(rpw // _L)
                ]
                for c, m, v in vals:
                    blk_v[buf, c, pl.ds(m * _L, _L)] = v
            pltpu.async_copy(
                blk_v.at[buf], out_t.at[j, :, pl.ds(bbase, rpw)], sem
            )
            return carry

        lax.fori_loop(0, n, j_body, 0)
        for _ in range(_NBUF):
            drain_one()

    return sc_lookup


def kernel(numbers, table):
    b, n = numbers.shape
    _, d = table.shape
    x_t = jnp.transpose(numbers).reshape(n, b // 128, 128)
    rank_t = _rank_call(x_t)
    rank1 = jnp.transpose(rank_t.reshape(n, b)).reshape(b * n)
    tbl1 = table[:n].reshape(n * d)
    out_t = _make_sc_lookup(b, n, d)(rank1, tbl1)
    return jnp.transpose(out_t, (2, 0, 1))
